# baseline (speedup 1.0000x reference)
import functools

import jax
import jax.numpy as jnp
from jax import lax
from jax.experimental import pallas as pl
from jax.experimental.pallas import tpu as pltpu
from jax.experimental.pallas import tpu_sc as plsc

_EMBED = 64
_OUT_D = 192
_PAD_D = 128
_NC, _NS = 2, 16
_NW = _NC * _NS
_BLK = 128
_H = 64
_SUP = 8
_SUPT = _SUP * _BLK


def _build(B: int, L: int):
    n_tokens = B * L
    n_blocks = n_tokens // _BLK
    blk_per_w = n_blocks // _NW
    n_sup = blk_per_w // _SUP
    n_q = n_sup // 2
    assert n_blocks % _NW == 0 and blk_per_w % _SUP == 0 and n_sup % 2 == 0
    assert B % _BLK == 0
    bblk = B // _BLK

    mesh = plsc.VectorSubcoreMesh(
        core_axis_name="c", subcore_axis_name="s",
        num_cores=_NC, num_subcores=_NS,
    )

    @functools.partial(
        pl.kernel,
        out_type=jax.ShapeDtypeStruct((L, _OUT_D, B), jnp.float32),
        mesh=mesh,
        scratch_types=[
            pltpu.VMEM((2 * 3 * _SUPT,), jnp.int32),
            pltpu.VMEM((_H, _PAD_D), jnp.float32),
            pltpu.VMEM((_H, _PAD_D), jnp.float32),
            pltpu.VMEM((_H, _PAD_D), jnp.float32),
            pltpu.VMEM((_H, _PAD_D), jnp.float32),
            pltpu.VMEM((_H, _PAD_D), jnp.float32),
            pltpu.VMEM((_H, _PAD_D), jnp.float32),
            pltpu.VMEM((_OUT_D, _BLK), jnp.float32),
            pltpu.VMEM((_OUT_D, _BLK), jnp.float32),
            pltpu.SemaphoreType.DMA,
            pltpu.SemaphoreType.DMA,
            pltpu.SemaphoreType.DMA,
            pltpu.SemaphoreType.DMA,
            pltpu.SemaphoreType.DMA,
            pltpu.SemaphoreType.DMA,
        ],
        compiler_params=pltpu.CompilerParams(needs_layout_passes=False),
    )
    def run(ner_i, dep_i, pner_i, ner_t, dep_t, pner_t, out,
            idx_v, nr0, nr1, dr0, dr1, pr0, pr1, tr0, tr1,
            gsem0, gsem1, wsem0, wsem1, isem0, isem1):
        wid = lax.axis_index("s") * _NC + lax.axis_index("c")
        tok_w = wid * (blk_per_w * _BLK)
        blk_w = wid * blk_per_w
        tabs = (ner_t, dep_t, pner_t)
        srcs = (ner_i, dep_i, pner_i)
        rows = ((nr0, nr1), (dr0, dr1), (pr0, pr1))
        trs = (tr0, tr1)
        gsem = (gsem0, gsem1)
        wsem = (wsem0, wsem1)
        isem = (isem0, isem1)

        def fire_idx(sup_idx, islot, sem):
            for t, src in enumerate(srcs):
                pltpu.async_copy(
                    src.at[pl.ds(tok_w + sup_idx * _SUPT, _SUPT)],
                    idx_v.at[pl.ds((islot * 3 + t) * _SUPT, _SUPT)], sem)

        def wait_idx(islot, sem):
            for t in range(3):
                pltpu.make_async_copy(
                    ner_i.at[pl.ds(tok_w, _SUPT)],
                    idx_v.at[pl.ds((islot * 3 + t) * _SUPT, _SUPT)],
                    sem).wait()

        def fire_g(b, islot, off):
            for t, tab in enumerate(tabs):
                pltpu.async_copy(
                    tab.at[idx_v.at[pl.ds((islot * 3 + t) * _SUPT + off, _H)]],
                    rows[t][b], gsem[b])

        def wait_g(b):
            for t in range(3):
                pltpu.make_async_copy(
                    ner_t.at[pl.ds(0, _H)], rows[t][b], gsem[b]).wait()

        def fire_w(gb, blk):
            l = lax.shift_right_logical(blk, 7)
            b0 = pl.multiple_of(
                lax.shift_left(lax.bitwise_and(blk, bblk - 1), 7), _BLK)
            pltpu.async_copy(
                trs[gb], out.at[l, :, pl.ds(b0, _BLK)], wsem[gb])

        def wait_w(gb):
            pltpu.make_async_copy(
                trs[gb], out.at[0, :, pl.ds(0, _BLK)], wsem[gb]).wait()

        iota = lax.iota(jnp.int32, 16)
        row_i0 = [iota + 16 * i0 for i0 in range(_H // 16)]

        def transpose_half(b, gb, hb):
            tr_r = trs[gb]

            def body(f, _):
                fsp = jnp.zeros((16,), jnp.int32) + f
                for t in range(3):
                    src = rows[t][b]
                    for i0 in range(_H // 16):
                        val = plsc.load_gather(src, [row_i0[i0], fsp])
                        tr_r[t * _EMBED + f,
                             pl.ds(hb * _H + i0 * 16, 16)] = val
                return 0

            lax.fori_loop(0, _EMBED, body, 0)

        fire_idx(0, 0, isem[0])
        fire_idx(1, 1, isem[1])
        wait_idx(0, isem[0])
        fire_g(0, 0, 0)
        fire_g(1, 0, _H)

        def outer(q, _):
            for p in (0, 1):
                s = 2 * q + p
                for j2 in range(16):
                    b = j2 % 2
                    gb = (j2 // 2) % 2
                    hb = b
                    blk = s * _SUP + (j2 // 2)
                    wait_g(b)
                    if hb == 0:
                        if p == 0 and j2 < 4:
                            @pl.when(q > 0)
                            def _():
                                wait_w(gb)
                        else:
                            wait_w(gb)
                    transpose_half(b, gb, hb)
                    if hb == 1:
                        fire_w(gb, blk_w + blk)
                    if j2 < 14:
                        fire_g(b, p, ((j2 // 2) + 1) * _BLK + b * _H)
                    else:
                        if j2 == 14:
                            if p == 1:
                                @pl.when(q < n_q - 1)
                                def _():
                                    wait_idx(1 - p, isem[1 - p])
                            else:
                                wait_idx(1 - p, isem[1 - p])
                        if p == 1:
                            @pl.when(q < n_q - 1)
                            def _():
                                fire_g(b, 1 - p, b * _H)
                        else:
                            fire_g(b, 1 - p, b * _H)
                    if j2 == 15:
                        @pl.when(q < n_q - 1)
                        def _():
                            fire_idx(s + 2, p, isem[p])
            return 0

        lax.fori_loop(0, n_q, outer, 0)
        wait_w(0)
        wait_w(1)

    return run


def kernel(ner_ids, dep_ids, p_ner_ids, ner_table, dep_table, p_ner_table):
    B, L = ner_ids.shape
    n_tokens = B * L
    ids = [a.astype(jnp.int32).T.reshape(n_tokens)
           for a in (ner_ids, dep_ids, p_ner_ids)]
    pad = ((0, 0), (0, _PAD_D - _EMBED))
    tabs = [jnp.pad(t, pad) for t in (ner_table, dep_table, p_ner_table)]
    run = _build(B, L)
    out = run(*ids, *tabs)
    return out.transpose(2, 0, 1)


# speedup vs baseline: 2.2739x; 2.2739x over previous
import functools

import jax
import jax.numpy as jnp
from jax import lax
from jax.experimental import pallas as pl
from jax.experimental.pallas import tpu as pltpu
from jax.experimental.pallas import tpu_sc as plsc

_EMBED = 64
_OUT_D = 192
_PAD_D = 128
_NC, _NS = 2, 16
_NW = _NC * _NS
_C = 80
_SUP = 8
_SUPW = 3 * _C * _SUP


def _build(n_tokens: int):
    n_per_w = n_tokens // _NW
    n_chunks = n_per_w // _C
    n_sup = n_chunks // _SUP
    n_q = n_sup // 2
    assert n_per_w % _C == 0 and n_chunks % _SUP == 0 and n_sup % 2 == 0

    mesh = plsc.VectorSubcoreMesh(
        core_axis_name="c", subcore_axis_name="s",
        num_cores=_NC, num_subcores=_NS,
    )

    @functools.partial(
        pl.kernel,
        out_type=jax.ShapeDtypeStruct((n_tokens, _OUT_D), jnp.float32),
        mesh=mesh,
        scratch_types=[
            pltpu.VMEM((2 * _SUPW,), jnp.int32),
            pltpu.VMEM((2, _C, _PAD_D), jnp.float32),
            pltpu.VMEM((2, _C, _PAD_D), jnp.float32),
            pltpu.VMEM((2, _C, _PAD_D), jnp.float32),
            pltpu.VMEM((2, _C, _OUT_D), jnp.float32),
            pltpu.SemaphoreType.DMA,
            pltpu.SemaphoreType.DMA,
            pltpu.SemaphoreType.DMA,
            pltpu.SemaphoreType.DMA,
            pltpu.SemaphoreType.DMA,
            pltpu.SemaphoreType.DMA,
        ],
    )
    def run(ner_i, dep_i, pner_i, ner_t, dep_t, pner_t, out,
            idx_v, nr_v, dr_v, pr_v, big_v,
            gsem0, gsem1, wsem0, wsem1, isem0, isem1):
        wid = lax.axis_index("s") * _NC + lax.axis_index("c")
        base_w = wid * n_per_w
        tabs = (ner_t, dep_t, pner_t)
        srcs = (ner_i, dep_i, pner_i)
        gsem = (gsem0, gsem1)
        wsem = (wsem0, wsem1)
        isem = (isem0, isem1)
        supc = _SUP * _C

        def fire_idx(sup_idx, islot, sem):
            for t, src in enumerate(srcs):
                pltpu.async_copy(
                    src.at[pl.ds(base_w + sup_idx * supc, supc)],
                    idx_v.at[pl.ds((islot * 3 + t) * supc, supc)], sem)

        def wait_idx(islot, sem):
            for t in range(3):
                pltpu.make_async_copy(
                    ner_i.at[pl.ds(base_w, supc)],
                    idx_v.at[pl.ds((islot * 3 + t) * supc, supc)], sem).wait()

        def fire_g(b, islot, jj):
            for t, (tab, dst) in enumerate(zip(tabs, (nr_v, dr_v, pr_v))):
                pltpu.async_copy(
                    tab.at[idx_v.at[pl.ds((islot * 3 + t) * supc + jj * _C, _C)]],
                    dst.at[b], gsem[b])

        def wait_g(b):
            for dst in (nr_v, dr_v, pr_v):
                pltpu.make_async_copy(
                    ner_t.at[pl.ds(0, _C)], dst.at[b], gsem[b]).wait()

        def fire_w(b, base):
            pltpu.async_copy(big_v.at[b], out.at[pl.ds(base, _C)], wsem[b])

        def wait_w(b):
            pltpu.make_async_copy(
                big_v.at[b], out.at[pl.ds(base_w, _C)], wsem[b]).wait()

        fire_idx(0, 0, isem[0])
        fire_idx(1, 1, isem[1])
        wait_idx(0, isem[0])
        fire_g(0, 0, 0)
        fire_g(1, 0, 1)

        def outer(q, _):
            for p in (0, 1):
                s0 = 2 * q + p
                for j in range(8):
                    b = j % 2
                    k = s0 * 8 + j
                    base = base_w + k * _C
                    wait_g(b)
                    if p == 0 and j < 2:
                        @pl.when(q > 0)
                        def _():
                            wait_w(b)
                    else:
                        wait_w(b)

                    big_r = big_v.at[b]
                    nr_r, dr_r, pr_r = nr_v.at[b], dr_v.at[b], pr_v.at[b]

                    def intl(i2, _):
                        i = i2 * 2
                        for d in range(2):
                            for m in range(_EMBED // 16):
                                sl = pl.ds(m * 16, 16)
                                big_r[i + d, pl.ds(m * 16, 16)] = nr_r[i + d, sl]
                                big_r[i + d, pl.ds(_EMBED + m * 16, 16)] = dr_r[i + d, sl]
                                big_r[i + d, pl.ds(2 * _EMBED + m * 16, 16)] = pr_r[i + d, sl]
                        return 0

                    lax.fori_loop(0, _C // 2, intl, 0)
                    fire_w(b, base)

                    if j < 6:
                        fire_g(b, p, j + 2)
                    else:
                        if j == 6:
                            if p == 1:
                                @pl.when(q < n_q - 1)
                                def _():
                                    wait_idx(1 - p, isem[1 - p])
                            else:
                                wait_idx(1 - p, isem[1 - p])
                        if p == 1:
                            @pl.when(q < n_q - 1)
                            def _():
                                fire_g(b, 1 - p, j - 6)
                        else:
                            fire_g(b, 1 - p, j - 6)
                    if j == 7:
                        @pl.when(q < n_q - 1)
                        def _():
                            fire_idx(s0 + 2, p, isem[p])
            return 0

        lax.fori_loop(0, n_q, outer, 0)
        wait_w(0)
        wait_w(1)

    return run


def kernel(ner_ids, dep_ids, p_ner_ids, ner_table, dep_table, p_ner_table):
    B, L = ner_ids.shape
    n_tokens = B * L
    ids = [a.reshape(n_tokens).astype(jnp.int32)
           for a in (ner_ids, dep_ids, p_ner_ids)]
    pad = ((0, 0), (0, _PAD_D - _EMBED))
    tabs = [jnp.pad(t, pad) for t in (ner_table, dep_table, p_ner_table)]
    run = _build(n_tokens)
    out = run(*ids, *tabs)
    return out.reshape(B, L, _OUT_D)
